# baseline (speedup 1.0000x reference)
import functools

import jax
import jax.numpy as jnp
from jax import lax
from jax.experimental import pallas as pl
from jax.experimental.pallas import tpu as pltpu
from jax.experimental.pallas import tpu_sc as plsc

N = 16777216
NUM_CORES = 2
NUM_SUBCORES = 16
NW = NUM_CORES * NUM_SUBCORES
PER_W = N // NW
CHUNK = 16384
NCHUNK = PER_W // CHUNK
LANES = 16
NVEC = CHUNK // LANES


def _make_sc_add():
    mesh = plsc.VectorSubcoreMesh(core_axis_name="c", subcore_axis_name="s")

    @functools.partial(
        pl.kernel,
        mesh=mesh,
        out_type=jax.ShapeDtypeStruct((N,), jnp.float32),
        scratch_types=[
            pltpu.VMEM((2, CHUNK), jnp.float32),
            pltpu.VMEM((2, CHUNK), jnp.float32),
            pltpu.VMEM((2, CHUNK), jnp.float32),
            pltpu.SemaphoreType.DMA,
            pltpu.SemaphoreType.DMA,
            pltpu.SemaphoreType.DMA,
            pltpu.SemaphoreType.DMA,
            pltpu.SemaphoreType.DMA,
            pltpu.SemaphoreType.DMA,
        ],
    )
    def add_kernel(x_hbm, b_hbm, out_hbm, xv, bv, rv,
                   in_x0, in_x1, in_b0, in_b1, out0, out1):
        wid = lax.axis_index("s") * NUM_CORES + lax.axis_index("c")
        base = wid * PER_W
        in_x = (in_x0, in_x1)
        in_b = (in_b0, in_b1)
        out_s = (out0, out1)

        def start_in(s, g):
            off = base + g * CHUNK
            pltpu.async_copy(x_hbm.at[pl.ds(off, CHUNK)], xv.at[s], in_x[s])
            pltpu.async_copy(b_hbm.at[pl.ds(off, CHUNK)], bv.at[s], in_b[s])

        def wait_in(s, g):
            off = base + g * CHUNK
            pltpu.make_async_copy(x_hbm.at[pl.ds(off, CHUNK)], xv.at[s],
                                  in_x[s]).wait()
            pltpu.make_async_copy(b_hbm.at[pl.ds(off, CHUNK)], bv.at[s],
                                  in_b[s]).wait()

        def start_out(s, g):
            off = base + g * CHUNK
            pltpu.async_copy(rv.at[s], out_hbm.at[pl.ds(off, CHUNK)], out_s[s])

        def wait_out(s, g):
            off = base + g * CHUNK
            pltpu.make_async_copy(rv.at[s], out_hbm.at[pl.ds(off, CHUNK)],
                                  out_s[s]).wait()

        def compute(s):
            @plsc.parallel_loop(0, CHUNK, step=LANES, unroll=16)
            def add_body(i):
                sl = pl.ds(i, LANES)
                rv[s, sl] = xv[s, sl] + bv[s, sl]

        start_in(0, 0)
        start_in(1, 1)
        wait_in(0, 0)
        compute(0)
        start_out(0, 0)
        start_in(0, 2)
        wait_in(1, 1)
        compute(1)
        start_out(1, 1)
        start_in(1, 3)

        def pair_body(k, carry):
            g = 2 * k

            def step(s):
                gg = g + s
                wait_in(s, gg)
                wait_out(s, gg - 2)
                compute(s)
                start_out(s, gg)
                start_in(s, gg + 2)

            step(0)
            step(1)
            return carry

        lax.fori_loop(1, NCHUNK // 2 - 1, pair_body, 0)

        for s, g in ((0, NCHUNK - 2), (1, NCHUNK - 1)):
            wait_in(s, g)
            wait_out(s, g - 2)
            compute(s)
            start_out(s, g)
        wait_out(0, NCHUNK - 2)
        wait_out(1, NCHUNK - 1)

    return add_kernel


_sc_add = _make_sc_add()


def kernel(X, bias, out_idxs):
    del out_idxs
    return _sc_add(X, bias)


# speedup vs baseline: 1.8651x; 1.8651x over previous
import jax
import jax.numpy as jnp
from jax.experimental import pallas as pl

N = 16777216
BLK = 524288


def _tc_body(x_ref, b_ref, o_ref):
    o_ref[...] = x_ref[...] + b_ref[...]


def _tc_add(x, b):
    return pl.pallas_call(
        _tc_body,
        grid=(N // BLK,),
        in_specs=[
            pl.BlockSpec((BLK,), lambda i: (i,)),
            pl.BlockSpec((BLK,), lambda i: (i,)),
        ],
        out_specs=pl.BlockSpec((BLK,), lambda i: (i,)),
        out_shape=jax.ShapeDtypeStruct((N,), jnp.float32),
    )(x, b)


def kernel(X, bias, out_idxs):
    del out_idxs
    return _tc_add(X, bias)
